# Initial kernel scaffold; baseline (speedup 1.0000x reference)
import functools

import jax
import jax.numpy as jnp
from jax import lax
from jax.experimental import pallas as pl
from jax.experimental.pallas import tpu as pltpu
from jax.experimental.pallas import tpu_sc as plsc

NN = 10000
EE = 160000
GG = 64
NB = 400
EB = 2000
SB = 128
NTILE = 32
STRIPE = NN // 16
ZR = 125
F32 = jnp.float32


def _dot(a, b):
  return jnp.dot(a, b, preferred_element_type=F32)


def _prep_body(Wem2, bem2, We1, be1, We2, be2, We3, be3,
               W1o, b1o, W2o, b2o, W3o, b3o):
  for Wi, bi, Wo, bo in ((We1, be1, W1o, b1o), (We2, be2, W2o, b2o),
                         (We3, be3, W3o, b3o)):
    Wo[...] = _dot(Wem2[...], Wi[...])
    bo[...] = _dot(bem2[...], Wi[...]) + bi[...]


def _prep(Wem2, bem2r, We1p, be1pr, We2, be2r, We3, be3r):
  outs = [jax.ShapeDtypeStruct((64, 16), F32), jax.ShapeDtypeStruct((1, 16), F32),
          jax.ShapeDtypeStruct((64, 512), F32), jax.ShapeDtypeStruct((1, 512), F32),
          jax.ShapeDtypeStruct((64, 1024), F32), jax.ShapeDtypeStruct((1, 1024), F32)]
  return pl.pallas_call(_prep_body, out_shape=outs)(
      Wem2, bem2r, We1p, be1pr, We2, be2r, We3, be3r)


def _edge_body(ea, Wm, bm, W1, b1, W2, b2, W3, b3, e1, *eouts):
  a = jnp.maximum(_dot(ea[...], Wm[...]) + bm[...], 0.0)
  e1[...] = _dot(a, W1[...]) + b1[...]
  W2v = W2[...]
  b2v = b2[...]
  for c in range(4):
    s = slice(c * 128, (c + 1) * 128)
    eouts[c][...] = _dot(a, W2v[:, s]) + b2v[:, s]
  W3v = W3[...]
  b3v = b3[...]
  for c in range(8):
    s = slice(c * 128, (c + 1) * 128)
    eouts[4 + c][...] = _dot(a, W3v[:, s]) + b3v[:, s]


def _edge(ea_p, Wem1p, bem1r, W21, b21, W22, b22, W23, b23):
  n = EE // EB
  fullw = [
      pl.BlockSpec(s, lambda i: (0, 0))
      for s in [(8, 64), (1, 64), (64, 16), (1, 16), (64, 512), (1, 512),
                (64, 1024), (1, 1024)]
  ]
  outs = ([jax.ShapeDtypeStruct((EE, 16), F32)] +
          [jax.ShapeDtypeStruct((EE, 128), F32) for _ in range(12)])
  out_specs = ([pl.BlockSpec((EB, 16), lambda i: (i, 0))] +
               [pl.BlockSpec((EB, 128), lambda i: (i, 0)) for _ in range(12)])
  res = pl.pallas_call(
      _edge_body,
      grid=(n,),
      in_specs=[pl.BlockSpec((EB, 8), lambda i: (i, 0))] + fullw,
      out_specs=out_specs,
      out_shape=outs,
  )(ea_p, Wem1p, bem1r, W21, b21, W22, b22, W23, b23)
  return res[0], res[1:5], res[5:13]


def _make_sc_stage(C, W):
  mesh = plsc.VectorSubcoreMesh(core_axis_name="c", subcore_axis_name="s")
  nblocks = EE // SB
  out_type = [jax.ShapeDtypeStruct((2, NN, W), F32) for _ in range(C)]
  scratch = [
      pltpu.VMEM_SHARED((NN, W), F32),
      pltpu.VMEM((SB,), jnp.int32),
      pltpu.VMEM((SB,), jnp.int32),
      pltpu.VMEM((SB, W), F32),
      pltpu.VMEM((SB, W), F32),
      pltpu.VMEM((ZR, W), F32),
      pltpu.SemaphoreType.DMA,
  ]

  @functools.partial(pl.kernel, out_type=out_type, mesh=mesh,
                     scratch_types=scratch)
  def k(*refs):
    h_refs = refs[:C]
    e_refs = refs[C:2 * C]
    src_hbm = refs[2 * C]
    dst_hbm = refs[2 * C + 1]
    out_refs = refs[2 * C + 2:3 * C + 2]
    slab, src_v, dst_v, rows_v, e_v, zero_v, sem = refs[3 * C + 2:]
    core = lax.axis_index("c")
    sub = lax.axis_index("s")
    wid = core * 16 + sub

    zvec = jnp.zeros((16,), F32)

    def zr(rr, carry):
      for kk in range(W // 16):
        zero_v[rr, pl.ds(kk * 16, 16)] = zvec
      return carry

    lax.fori_loop(0, ZR, zr, 0)

    nb_t = (nblocks - wid + NTILE - 1) // NTILE

    for c in range(C):
      for kk in range(STRIPE // ZR):
        pltpu.sync_copy(zero_v, slab.at[pl.ds(sub * STRIPE + kk * ZR, ZR)])
      plsc.subcore_barrier()

      def eb(j, carry):
        off = (wid + j * NTILE) * SB
        pltpu.sync_copy(src_hbm.at[pl.ds(off, SB)], src_v)
        pltpu.sync_copy(dst_hbm.at[pl.ds(off, SB)], dst_v)
        pltpu.async_copy(h_refs[c].at[src_v], rows_v, sem).wait()
        pltpu.sync_copy(e_refs[c].at[pl.ds(off, SB)], e_v)

        def rb(rr, rc):
          for kk in range(W // 16):
            s = pl.ds(kk * 16, 16)
            rows_v[rr, s] = jnp.maximum(rows_v[rr, s] + e_v[rr, s], 0.0)
          return rc

        lax.fori_loop(0, SB, rb, 0)
        pltpu.sync_copy(rows_v, slab.at[dst_v], add=True)
        return carry

      lax.fori_loop(0, nb_t, eb, 0)
      plsc.subcore_barrier()
      pltpu.sync_copy(slab.at[pl.ds(sub * STRIPE, STRIPE)],
                      out_refs[c].at[core, pl.ds(sub * STRIPE, STRIPE)])

  return k


_sc_stage_1 = _make_sc_stage(1, 16)
_sc_stage_2 = _make_sc_stage(4, 128)
_sc_stage_3 = _make_sc_stage(8, 128)


def _make_node(Cin, Win, Fout):
  n = NN // NB

  def body(*refs):
    h = refs[:Cin]
    p = refs[Cin:2 * Cin]
    Wn = refs[2 * Cin]
    nb_ = refs[2 * Cin + 1]
    y, ssum, ssq = refs[2 * Cin + 2:]
    i = pl.program_id(0)
    Wv = Wn[...]
    z = jnp.broadcast_to(nb_[...], (NB, Fout))
    for c in range(Cin):
      pv = p[c][...]
      hc = h[c][...] + pv[0] + pv[1]
      z = z + _dot(hc, Wv[c * Win:(c + 1) * Win, :])
    yv = jnp.maximum(z, 0.0)
    y[...] = yv

    @pl.when(i == 0)
    def _():
      ssum[...] = jnp.zeros((1, Fout), F32)
      ssq[...] = jnp.zeros((1, Fout), F32)

    ssum[...] += jnp.sum(yv, axis=0, keepdims=True)
    ssq[...] += jnp.sum(yv * yv, axis=0, keepdims=True)

  in_specs = ([pl.BlockSpec((NB, Win), lambda i: (i, 0)) for _ in range(Cin)] +
              [pl.BlockSpec((2, NB, Win), lambda i: (0, i, 0)) for _ in range(Cin)] +
              [pl.BlockSpec((Cin * Win, Fout), lambda i: (0, 0)),
               pl.BlockSpec((1, Fout), lambda i: (0, 0))])
  out_specs = [pl.BlockSpec((NB, Fout), lambda i: (i, 0)),
               pl.BlockSpec((1, Fout), lambda i: (0, 0)),
               pl.BlockSpec((1, Fout), lambda i: (0, 0))]
  outs = [jax.ShapeDtypeStruct((NN, Fout), F32),
          jax.ShapeDtypeStruct((1, Fout), F32),
          jax.ShapeDtypeStruct((1, Fout), F32)]
  return pl.pallas_call(body, grid=(n,), in_specs=in_specs,
                        out_specs=out_specs, out_shape=outs)


_node_1 = _make_node(1, 16, 512)
_node_2 = _make_node(4, 128, 1024)
_node_3 = _make_node(8, 128, 2048)


def _make_norm(Fout):
  Cout = Fout // 128
  n = NN // NB

  def body(y, ssum, ssq, g, b, *outs):
    m = ssum[...] / NN
    v = ssq[...] / NN - m * m
    scale = lax.rsqrt(v + 1e-5) * g[...]
    yv = y[...]
    bv = b[...]
    for c in range(Cout):
      s = slice(c * 128, (c + 1) * 128)
      outs[c][...] = (yv[:, s] - m[:, s]) * scale[:, s] + bv[:, s]

  in_specs = [pl.BlockSpec((NB, Fout), lambda i: (i, 0)),
              pl.BlockSpec((1, Fout), lambda i: (0, 0)),
              pl.BlockSpec((1, Fout), lambda i: (0, 0)),
              pl.BlockSpec((1, Fout), lambda i: (0, 0)),
              pl.BlockSpec((1, Fout), lambda i: (0, 0))]
  out_specs = [pl.BlockSpec((NB, 128), lambda i: (i, 0)) for _ in range(Cout)]
  outs = [jax.ShapeDtypeStruct((NN, 128), F32) for _ in range(Cout)]
  return pl.pallas_call(body, grid=(n,), in_specs=in_specs,
                        out_specs=out_specs, out_shape=outs)


_norm_1 = _make_norm(512)
_norm_2 = _make_norm(1024)


def _final_body(y3, batchr, ssum, ssq, g3, b3, Wf1, bf1, Wf2, bf2, Wf3, bf3,
                out, Sacc, Cacc):
  i = pl.program_id(0)

  @pl.when(i == 0)
  def _():
    Sacc[...] = jnp.zeros((GG, 2048), F32)
    Cacc[...] = jnp.zeros((GG, 128), F32)

  bvec = batchr[0, 0, :]
  gid = lax.broadcasted_iota(jnp.int32, (GG, NB), 0)
  M = (bvec[None, :] == gid).astype(F32)
  Sacc[...] += _dot(M, y3[...])
  Cacc[...] += jnp.sum(M, axis=1, keepdims=True)

  @pl.when(i == (NN // NB) - 1)
  def _():
    m = ssum[...] / NN
    v = ssq[...] / NN - m * m
    r = lax.rsqrt(v + 1e-5)
    cnt = Cacc[:, 0:1]
    ph = (Sacc[...] - cnt * m) * (r * g3[...]) + cnt * b3[...]
    pooled = ph / jnp.maximum(cnt, 1.0)
    t = jnp.maximum(_dot(pooled, Wf1[...]) + bf1[...], 0.0)
    t = jnp.maximum(_dot(t, Wf2[...]) + bf2[...], 0.0)
    out[...] = _dot(t, Wf3[...]) + bf3[...]


def _final(y3, batchr, s3, q3, g3r, b3r, Wf1, bf1r, Wf2, bf2r, Wf3, bf3r):
  n = NN // NB
  in_specs = [pl.BlockSpec((NB, 2048), lambda i: (i, 0)),
              pl.BlockSpec((1, 1, NB), lambda i: (i, 0, 0))] + [
      pl.BlockSpec(s, lambda i: (0, 0))
      for s in [(1, 2048), (1, 2048), (1, 2048), (1, 2048),
                (2048, 1024), (1, 1024), (1024, 512), (1, 512),
                (512, 86), (1, 86)]
  ]
  return pl.pallas_call(
      _final_body,
      grid=(n,),
      in_specs=in_specs,
      out_specs=pl.BlockSpec((GG, 86), lambda i: (0, 0)),
      out_shape=jax.ShapeDtypeStruct((GG, 86), F32),
      scratch_shapes=[pltpu.VMEM((GG, 2048), F32), pltpu.VMEM((GG, 128), F32)],
  )(y3, batchr, s3, q3, g3r, b3r, Wf1, bf1r, Wf2, bf2r, Wf3, bf3r)


def kernel(x, edge_index, edge_attr, batch, Wem1, bem1, Wem2, bem2, We1, be1,
           Wn1, nb1, g1, b1, We2, be2, Wn2, nb2, g2, b2, We3, be3, Wn3, nb3,
           g3, b3, Wf1, bf1, Wf2, bf2, Wf3, bf3):
  src = edge_index[0]
  dst = edge_index[1]
  r = lambda v: v.reshape(1, -1)

  ea_p = jnp.pad(edge_attr, ((0, 0), (0, 2)))
  Wem1p = jnp.pad(Wem1, ((0, 2), (0, 0)))
  We1p = jnp.pad(We1, ((0, 0), (0, 10)))
  be1p = jnp.pad(be1, (0, 10))
  x_p = jnp.pad(x, ((0, 0), (0, 10)))
  Wn1p = jnp.pad(Wn1, ((0, 10), (0, 0)))
  batchr = batch.reshape(NN // NB, 1, NB)

  W21, b21, W22, b22, W23, b23 = _prep(
      Wem2, r(bem2), We1p, r(be1p), We2, r(be2), We3, r(be3))
  e1, e2c, e3c = _edge(ea_p, Wem1p, r(bem1), W21, b21, W22, b22, W23, b23)

  p1 = _sc_stage_1(x_p, e1, src, dst)
  y1, s1, q1 = _node_1(x_p, p1[0], Wn1p, r(nb1))
  h1c = _norm_1(y1, s1, q1, r(g1), r(b1))

  p2 = _sc_stage_2(*h1c, *e2c, src, dst)
  y2, s2, q2 = _node_2(*h1c, *p2, Wn2, r(nb2))
  h2c = _norm_2(y2, s2, q2, r(g2), r(b2))

  p3 = _sc_stage_3(*h2c, *e3c, src, dst)
  y3, s3, q3 = _node_3(*h2c, *p3, Wn3, r(nb3))

  return _final(y3, batchr, s3, q3, r(g3), r(b3), Wf1, r(bf1), Wf2, r(bf2),
                Wf3, r(bf3))


# speedup vs baseline: 1.5119x; 1.5119x over previous
import functools

import jax
import jax.numpy as jnp
from jax import lax
from jax.experimental import pallas as pl
from jax.experimental.pallas import tpu as pltpu
from jax.experimental.pallas import tpu_sc as plsc

NN = 10000
EE = 160000
GG = 64
NB = 400
EB = 2000
SB = 128
NTILE = 32
STRIPE = 640
ZR = 80
F32 = jnp.float32


def _dot(a, b):
  return jnp.dot(a.astype(jnp.bfloat16), b.astype(jnp.bfloat16),
                 preferred_element_type=F32)


def _dotx(a, b):
  return jnp.dot(a, b, preferred_element_type=F32,
                 precision=lax.Precision.HIGHEST)


def _edge_body(ea, Wm, bm, Wm2, bm2, W1, b1, W2, b2, W3, b3, e1, *eouts):
  a1 = jnp.maximum(_dot(ea[...], Wm[...]) + bm[...], 0.0)
  a = _dot(a1, Wm2[...]) + bm2[...]
  e1[...] = _dot(a, W1[...]) + b1[...]
  W2v = W2[...]
  b2v = b2[...]
  for c in range(4):
    s = slice(c * 128, (c + 1) * 128)
    eouts[c][...] = _dot(a, W2v[:, s]) + b2v[:, s]
  W3v = W3[...]
  b3v = b3[...]
  for c in range(8):
    s = slice(c * 128, (c + 1) * 128)
    eouts[4 + c][...] = _dot(a, W3v[:, s]) + b3v[:, s]


def _edge(ea_p, Wem1p, bem1r, Wem2, bem2r, We1p, be1pr, We2, be2r, We3, be3r):
  n = EE // EB
  fullw = [
      pl.BlockSpec(s, lambda i: (0, 0))
      for s in [(8, 64), (1, 64), (64, 64), (1, 64), (64, 128), (1, 128),
                (64, 512), (1, 512), (64, 1024), (1, 1024)]
  ]
  outs = [jax.ShapeDtypeStruct((EE, 128), F32) for _ in range(13)]
  out_specs = [pl.BlockSpec((EB, 128), lambda i: (i, 0)) for _ in range(13)]
  res = pl.pallas_call(
      _edge_body,
      grid=(n,),
      in_specs=[pl.BlockSpec((EB, 8), lambda i: (i, 0))] + fullw,
      out_specs=out_specs,
      out_shape=outs,
  )(ea_p, Wem1p, bem1r, Wem2, bem2r, We1p, be1pr, We2, be2r, We3, be3r)
  return res[0], res[1:5], res[5:13]


def _make_sc_stage(C, W):
  mesh = plsc.VectorSubcoreMesh(core_axis_name="c", subcore_axis_name="s")
  nblocks = EE // SB
  out_type = [jax.ShapeDtypeStruct((2, NN, W), F32) for _ in range(C)]
  scratch = [
      pltpu.VMEM_SHARED((NN, W), F32),
      pltpu.VMEM((SB,), jnp.int32),
      pltpu.VMEM((SB,), jnp.int32),
      pltpu.VMEM((SB, W), F32),
      pltpu.VMEM((SB, W), F32),
      pltpu.VMEM((ZR, W), F32),
      pltpu.SemaphoreType.DMA,
  ]

  @functools.partial(pl.kernel, out_type=out_type, mesh=mesh,
                     scratch_types=scratch)
  def k(*refs):
    h_refs = refs[:C]
    e_refs = refs[C:2 * C]
    src_hbm = refs[2 * C]
    dst_hbm = refs[2 * C + 1]
    out_refs = refs[2 * C + 2:3 * C + 2]
    slab, src_v, dst_v, rows_v, e_v, zero_v, sem = refs[3 * C + 2:]
    core = lax.axis_index("c")
    sub = lax.axis_index("s")
    wid = core * 16 + sub

    zvec = jnp.zeros((16,), F32)

    def zr(rr, carry):
      for kk in range(W // 16):
        zero_v[rr, pl.ds(kk * 16, 16)] = zvec
      return carry

    lax.fori_loop(0, ZR, zr, 0)

    nb_t = (nblocks - wid + NTILE - 1) // NTILE

    base = sub * STRIPE
    nzero = jnp.where(sub == 15, 5, 8)

    for c in range(C):

      def zb(kk, carry):
        pltpu.sync_copy(zero_v, slab.at[pl.ds(base + kk * ZR, ZR)])
        return carry

      lax.fori_loop(0, nzero, zb, 0)
      plsc.subcore_barrier()

      def eb(j, carry):
        off = (wid + j * NTILE) * SB
        pltpu.sync_copy(src_hbm.at[pl.ds(off, SB)], src_v)
        pltpu.sync_copy(dst_hbm.at[pl.ds(off, SB)], dst_v)
        pltpu.async_copy(h_refs[c].at[src_v], rows_v, sem).wait()
        pltpu.sync_copy(e_refs[c].at[pl.ds(off, SB)], e_v)

        def rb(rr, rc):
          for kk in range(W // 16):
            s = pl.ds(kk * 16, 16)
            rows_v[rr, s] = jnp.maximum(rows_v[rr, s] + e_v[rr, s], 0.0)
          return rc

        lax.fori_loop(0, SB, rb, 0)
        pltpu.sync_copy(rows_v, slab.at[dst_v], add=True)
        return carry

      lax.fori_loop(0, nb_t, eb, 0)
      plsc.subcore_barrier()

      @pl.when(sub < 15)
      def _():
        pltpu.sync_copy(slab.at[pl.ds(base, STRIPE)],
                        out_refs[c].at[core, pl.ds(base, STRIPE)])

      @pl.when(sub == 15)
      def _():
        pltpu.sync_copy(slab.at[pl.ds(base, NN - 15 * STRIPE)],
                        out_refs[c].at[core, pl.ds(base, NN - 15 * STRIPE)])

  return k


_sc_stage_1 = _make_sc_stage(1, 128)
_sc_stage_2 = _make_sc_stage(4, 128)
_sc_stage_3 = _make_sc_stage(8, 128)


def _make_node(Cin, Win, Fout):
  n = NN // NB

  def body(*refs):
    h = refs[:Cin]
    p = refs[Cin:2 * Cin]
    Wn = refs[2 * Cin]
    nb_ = refs[2 * Cin + 1]
    y, ssum, ssq = refs[2 * Cin + 2:]
    i = pl.program_id(0)
    Wv = Wn[...]
    z = jnp.broadcast_to(nb_[...], (NB, Fout))
    for c in range(Cin):
      pv = p[c][...]
      hc = h[c][...] + pv[0] + pv[1]
      z = z + _dot(hc, Wv[c * Win:(c + 1) * Win, :])
    yv = jnp.maximum(z, 0.0)
    y[...] = yv

    @pl.when(i == 0)
    def _():
      ssum[...] = jnp.zeros((1, Fout), F32)
      ssq[...] = jnp.zeros((1, Fout), F32)

    ssum[...] += jnp.sum(yv, axis=0, keepdims=True)
    ssq[...] += jnp.sum(yv * yv, axis=0, keepdims=True)

  in_specs = ([pl.BlockSpec((NB, Win), lambda i: (i, 0)) for _ in range(Cin)] +
              [pl.BlockSpec((2, NB, Win), lambda i: (0, i, 0)) for _ in range(Cin)] +
              [pl.BlockSpec((Cin * Win, Fout), lambda i: (0, 0)),
               pl.BlockSpec((1, Fout), lambda i: (0, 0))])
  out_specs = [pl.BlockSpec((NB, Fout), lambda i: (i, 0)),
               pl.BlockSpec((1, Fout), lambda i: (0, 0)),
               pl.BlockSpec((1, Fout), lambda i: (0, 0))]
  outs = [jax.ShapeDtypeStruct((NN, Fout), F32),
          jax.ShapeDtypeStruct((1, Fout), F32),
          jax.ShapeDtypeStruct((1, Fout), F32)]
  return pl.pallas_call(body, grid=(n,), in_specs=in_specs,
                        out_specs=out_specs, out_shape=outs)


_node_1 = _make_node(1, 128, 512)
_node_2 = _make_node(4, 128, 1024)
_node_3 = _make_node(8, 128, 2048)


def _make_norm(Fout):
  Cout = Fout // 128
  n = NN // NB

  def body(y, ssum, ssq, g, b, *outs):
    m = ssum[...] / NN
    v = ssq[...] / NN - m * m
    scale = (1.0 / jnp.sqrt(v + 1e-5)) * g[...]
    yv = y[...]
    bv = b[...]
    for c in range(Cout):
      s = slice(c * 128, (c + 1) * 128)
      outs[c][...] = (yv[:, s] - m[:, s]) * scale[:, s] + bv[:, s]

  in_specs = [pl.BlockSpec((NB, Fout), lambda i: (i, 0)),
              pl.BlockSpec((1, Fout), lambda i: (0, 0)),
              pl.BlockSpec((1, Fout), lambda i: (0, 0)),
              pl.BlockSpec((1, Fout), lambda i: (0, 0)),
              pl.BlockSpec((1, Fout), lambda i: (0, 0))]
  out_specs = [pl.BlockSpec((NB, 128), lambda i: (i, 0)) for _ in range(Cout)]
  outs = [jax.ShapeDtypeStruct((NN, 128), F32) for _ in range(Cout)]
  return pl.pallas_call(body, grid=(n,), in_specs=in_specs,
                        out_specs=out_specs, out_shape=outs)


_norm_1 = _make_norm(512)
_norm_2 = _make_norm(1024)


def _final_body(y3, batchr, ssum, ssq, g3, b3, Wf1, bf1, Wf2, bf2, Wf3, bf3,
                out, Sacc, Cacc):
  i = pl.program_id(0)

  @pl.when(i == 0)
  def _():
    Sacc[...] = jnp.zeros((GG, 2048), F32)
    Cacc[...] = jnp.zeros((GG, 128), F32)

  bvec = batchr[0, 0, :]
  gid = lax.broadcasted_iota(jnp.int32, (GG, NB), 0)
  M = (bvec[None, :] == gid).astype(F32)
  Sacc[...] += _dotx(M, y3[...])
  Cacc[...] += jnp.sum(M, axis=1, keepdims=True)

  @pl.when(i == (NN // NB) - 1)
  def _():
    m = ssum[...] / NN
    v = ssq[...] / NN - m * m
    r = 1.0 / jnp.sqrt(v + 1e-5)
    cnt = Cacc[:, 0:1]
    ph = (Sacc[...] - cnt * m) * (r * g3[...]) + cnt * b3[...]
    pooled = ph / jnp.maximum(cnt, 1.0)
    t = jnp.maximum(_dot(pooled, Wf1[...]) + bf1[...], 0.0)
    t = jnp.maximum(_dot(t, Wf2[...]) + bf2[...], 0.0)
    out[...] = _dot(t, Wf3[...]) + bf3[...]


def _final(y3, batchr, s3, q3, g3r, b3r, Wf1, bf1r, Wf2, bf2r, Wf3, bf3r):
  n = NN // NB
  in_specs = [pl.BlockSpec((NB, 2048), lambda i: (i, 0)),
              pl.BlockSpec((1, 1, NB), lambda i: (i, 0, 0))] + [
      pl.BlockSpec(s, lambda i: (0, 0))
      for s in [(1, 2048), (1, 2048), (1, 2048), (1, 2048),
                (2048, 1024), (1, 1024), (1024, 512), (1, 512),
                (512, 86), (1, 86)]
  ]
  return pl.pallas_call(
      _final_body,
      grid=(n,),
      in_specs=in_specs,
      out_specs=pl.BlockSpec((GG, 86), lambda i: (0, 0)),
      out_shape=jax.ShapeDtypeStruct((GG, 86), F32),
      scratch_shapes=[pltpu.VMEM((GG, 2048), F32), pltpu.VMEM((GG, 128), F32)],
  )(y3, batchr, s3, q3, g3r, b3r, Wf1, bf1r, Wf2, bf2r, Wf3, bf3r)


def kernel(x, edge_index, edge_attr, batch, Wem1, bem1, Wem2, bem2, We1, be1,
           Wn1, nb1, g1, b1, We2, be2, Wn2, nb2, g2, b2, We3, be3, Wn3, nb3,
           g3, b3, Wf1, bf1, Wf2, bf2, Wf3, bf3):
  src = edge_index[0]
  dst = edge_index[1]
  r = lambda v: v.reshape(1, -1)

  ea_p = jnp.pad(edge_attr, ((0, 0), (0, 2)))
  Wem1p = jnp.pad(Wem1, ((0, 2), (0, 0)))
  We1p = jnp.pad(We1, ((0, 0), (0, 122)))
  be1p = jnp.pad(be1, (0, 122))
  x_p = jnp.pad(x, ((0, 0), (0, 122)))
  Wn1p = jnp.pad(Wn1, ((0, 122), (0, 0)))
  batchr = batch.reshape(NN // NB, 1, NB)

  e1, e2c, e3c = _edge(ea_p, Wem1p, r(bem1), Wem2, r(bem2), We1p, r(be1p),
                       We2, r(be2), We3, r(be3))

  p1 = _sc_stage_1(x_p, e1, src, dst)
  y1, s1, q1 = _node_1(x_p, p1[0], Wn1p, r(nb1))
  h1c = _norm_1(y1, s1, q1, r(g1), r(b1))

  p2 = _sc_stage_2(*h1c, *e2c, src, dst)
  y2, s2, q2 = _node_2(*h1c, *p2, Wn2, r(nb2))
  h2c = _norm_2(y2, s2, q2, r(g2), r(b2))

  p3 = _sc_stage_3(*h2c, *e3c, src, dst)
  y3, s3, q3 = _node_3(*h2c, *p3, Wn3, r(nb3))

  return _final(y3, batchr, s3, q3, r(g3), r(b3), Wf1, r(bf1), Wf2, r(bf2),
                Wf3, r(bf3))


# speedup vs baseline: 2.3330x; 1.5431x over previous
import functools

import jax
import jax.numpy as jnp
from jax import lax
from jax.experimental import pallas as pl
from jax.experimental.pallas import tpu as pltpu
from jax.experimental.pallas import tpu_sc as plsc

NN = 10000
EE = 160000
GG = 64
NB = 400
EB = 2000
SB = 64
NTILE = 32
STRIPE = 640
ZR = 80
F32 = jnp.float32


def _dot(a, b):
  return jnp.dot(a.astype(jnp.bfloat16), b.astype(jnp.bfloat16),
                 preferred_element_type=F32)


def _dotx(a, b):
  return jnp.dot(a, b, preferred_element_type=F32,
                 precision=lax.Precision.HIGHEST)


def _edge_body(ea, Wm, bm, Wm2, bm2, W1, b1, W2, b2, W3, b3, e1, *eouts):
  a1 = jnp.maximum(_dot(ea[...], Wm[...]) + bm[...], 0.0)
  a = _dot(a1, Wm2[...]) + bm2[...]
  e1[...] = _dot(a, W1[...]) + b1[...]
  W2v = W2[...]
  b2v = b2[...]
  for c in range(4):
    s = slice(c * 128, (c + 1) * 128)
    eouts[c][...] = _dot(a, W2v[:, s]) + b2v[:, s]
  W3v = W3[...]
  b3v = b3[...]
  for c in range(8):
    s = slice(c * 128, (c + 1) * 128)
    eouts[4 + c][...] = _dot(a, W3v[:, s]) + b3v[:, s]


def _edge(ea_p, Wem1p, bem1r, Wem2, bem2r, We1p, be1pr, We2, be2r, We3, be3r):
  n = EE // EB
  fullw = [
      pl.BlockSpec(s, lambda i: (0, 0))
      for s in [(8, 64), (1, 64), (64, 64), (1, 64), (64, 128), (1, 128),
                (64, 512), (1, 512), (64, 1024), (1, 1024)]
  ]
  outs = [jax.ShapeDtypeStruct((EE, 128), F32) for _ in range(13)]
  out_specs = [pl.BlockSpec((EB, 128), lambda i: (i, 0)) for _ in range(13)]
  res = pl.pallas_call(
      _edge_body,
      grid=(n,),
      in_specs=[pl.BlockSpec((EB, 8), lambda i: (i, 0))] + fullw,
      out_specs=out_specs,
      out_shape=outs,
  )(ea_p, Wem1p, bem1r, Wem2, bem2r, We1p, be1pr, We2, be2r, We3, be3r)
  return res[0], res[1:5], res[5:13]


def _make_sc_stage(C, W):
  mesh = plsc.VectorSubcoreMesh(core_axis_name="c", subcore_axis_name="s")
  nblocks = EE // SB
  out_type = [jax.ShapeDtypeStruct((2, NN, W), F32) for _ in range(C)]
  bufset = [
      pltpu.VMEM((SB,), jnp.int32),
      pltpu.VMEM((SB,), jnp.int32),
      pltpu.VMEM((SB, W), F32),
      pltpu.VMEM((SB, W), F32),
      pltpu.SemaphoreType.DMA,
      pltpu.SemaphoreType.DMA,
  ]
  scratch = [
      pltpu.VMEM_SHARED((NN, W), F32),
  ] + bufset + bufset

  @functools.partial(pl.kernel, out_type=out_type, mesh=mesh,
                     scratch_types=scratch)
  def k(*refs):
    h_refs = refs[:C]
    e_refs = refs[C:2 * C]
    src_hbm = refs[2 * C]
    dst_hbm = refs[2 * C + 1]
    out_refs = refs[2 * C + 2:3 * C + 2]
    slab = refs[3 * C + 2]
    bufA = refs[3 * C + 3:3 * C + 9]
    bufB = refs[3 * C + 9:3 * C + 15]
    core = lax.axis_index("c")
    sub = lax.axis_index("s")
    wid = core * 16 + sub

    zvec = jnp.zeros((16,), F32)

    nb_t = (nblocks - wid + NTILE - 1) // NTILE
    blk_off = lambda j: (wid + j * NTILE) * SB

    def start_idx(S, off):
      pltpu.make_async_copy(src_hbm.at[pl.ds(off, SB)], S[0], S[4]).start()
      pltpu.make_async_copy(dst_hbm.at[pl.ds(off, SB)], S[1], S[4]).start()

    def wait_idx(S):
      pltpu.make_async_copy(src_hbm.at[pl.ds(0, SB)], S[0], S[4]).wait()
      pltpu.make_async_copy(dst_hbm.at[pl.ds(0, SB)], S[1], S[4]).wait()

    def start_ge(S, c, off):
      pltpu.make_async_copy(h_refs[c].at[S[0]], S[2], S[5]).start()
      pltpu.make_async_copy(e_refs[c].at[pl.ds(off, SB)], S[3], S[5]).start()

    def wait_ge(S, c):
      pltpu.make_async_copy(h_refs[c].at[S[0]], S[2], S[5]).wait()
      pltpu.make_async_copy(e_refs[c].at[pl.ds(0, SB)], S[3], S[5]).wait()

    base = sub * STRIPE
    nzero = jnp.where(sub == 15, 6, 10)

    for c in range(C):
      def zr(rr, carry):
        for kk in range(W // 16):
          bufA[2][rr, pl.ds(kk * 16, 16)] = zvec
        return carry

      lax.fori_loop(0, SB, zr, 0)

      def zb(kk, carry):
        pltpu.make_async_copy(bufA[2], slab.at[pl.ds(base + kk * SB, SB)],
                              bufA[4]).start()
        return carry

      lax.fori_loop(0, nzero, zb, 0)

      @pl.when(sub == 15)
      def _():
        pltpu.make_async_copy(bufA[2].at[pl.ds(0, 16)],
                              slab.at[pl.ds(base + 384, 16)], bufA[4]).start()

      def zw(kk, carry):
        pltpu.make_async_copy(bufA[2], slab.at[pl.ds(base, SB)],
                              bufA[4]).wait()
        return carry

      lax.fori_loop(0, nzero, zw, 0)

      @pl.when(sub == 15)
      def _():
        pltpu.make_async_copy(bufA[2].at[pl.ds(0, 16)],
                              slab.at[pl.ds(base, 16)], bufA[4]).wait()

      plsc.subcore_barrier()

      start_idx(bufA, blk_off(0))
      wait_idx(bufA)
      start_ge(bufA, c, blk_off(0))

      @pl.when(1 < nb_t)
      def _():
        start_idx(bufB, blk_off(1))

      def step(j, X, Y):
        wait_ge(X, c)

        @pl.when(j + 1 < nb_t)
        def _():
          wait_idx(Y)
          start_ge(Y, c, blk_off(j + 1))

        def rb(rr, rc):
          for dd in range(2):
            for kk in range(W // 16):
              s = pl.ds(kk * 16, 16)
              X[2][rr * 2 + dd, s] = jnp.maximum(
                  X[2][rr * 2 + dd, s] + X[3][rr * 2 + dd, s], 0.0)
          return rc

        lax.fori_loop(0, SB // 2, rb, 0)
        pltpu.sync_copy(X[2], slab.at[X[1]], add=True)

        @pl.when(j + 2 < nb_t)
        def _():
          start_idx(X, blk_off(j + 2))

      def eb(j, carry):

        @pl.when(j % 2 == 0)
        def _():
          step(j, bufA, bufB)

        @pl.when(j % 2 == 1)
        def _():
          step(j, bufB, bufA)

        return carry

      lax.fori_loop(0, nb_t, eb, 0)
      plsc.subcore_barrier()

      @pl.when(sub < 15)
      def _():
        pltpu.sync_copy(slab.at[pl.ds(base, STRIPE)],
                        out_refs[c].at[core, pl.ds(base, STRIPE)])

      @pl.when(sub == 15)
      def _():
        pltpu.sync_copy(slab.at[pl.ds(base, NN - 15 * STRIPE)],
                        out_refs[c].at[core, pl.ds(base, NN - 15 * STRIPE)])

  return k


_sc_stage_1 = _make_sc_stage(1, 128)
_sc_stage_2 = _make_sc_stage(4, 128)
_sc_stage_3 = _make_sc_stage(8, 128)


def _make_node(Cin, Win, Fout):
  n = NN // NB

  def body(*refs):
    h = refs[:Cin]
    p = refs[Cin:2 * Cin]
    Wn = refs[2 * Cin]
    nb_ = refs[2 * Cin + 1]
    y, ssum, ssq = refs[2 * Cin + 2:]
    i = pl.program_id(0)
    Wv = Wn[...]
    z = jnp.broadcast_to(nb_[...], (NB, Fout))
    for c in range(Cin):
      pv = p[c][...]
      hc = h[c][...] + pv[0] + pv[1]
      z = z + _dot(hc, Wv[c * Win:(c + 1) * Win, :])
    yv = jnp.maximum(z, 0.0)
    y[...] = yv

    @pl.when(i == 0)
    def _():
      ssum[...] = jnp.zeros((1, Fout), F32)
      ssq[...] = jnp.zeros((1, Fout), F32)

    ssum[...] += jnp.sum(yv, axis=0, keepdims=True)
    ssq[...] += jnp.sum(yv * yv, axis=0, keepdims=True)

  in_specs = ([pl.BlockSpec((NB, Win), lambda i: (i, 0)) for _ in range(Cin)] +
              [pl.BlockSpec((2, NB, Win), lambda i: (0, i, 0)) for _ in range(Cin)] +
              [pl.BlockSpec((Cin * Win, Fout), lambda i: (0, 0)),
               pl.BlockSpec((1, Fout), lambda i: (0, 0))])
  out_specs = [pl.BlockSpec((NB, Fout), lambda i: (i, 0)),
               pl.BlockSpec((1, Fout), lambda i: (0, 0)),
               pl.BlockSpec((1, Fout), lambda i: (0, 0))]
  outs = [jax.ShapeDtypeStruct((NN, Fout), F32),
          jax.ShapeDtypeStruct((1, Fout), F32),
          jax.ShapeDtypeStruct((1, Fout), F32)]
  return pl.pallas_call(body, grid=(n,), in_specs=in_specs,
                        out_specs=out_specs, out_shape=outs)


_node_1 = _make_node(1, 128, 512)
_node_2 = _make_node(4, 128, 1024)
_node_3 = _make_node(8, 128, 2048)


def _make_norm(Fout):
  Cout = Fout // 128
  n = NN // NB

  def body(y, ssum, ssq, g, b, *outs):
    m = ssum[...] / NN
    v = ssq[...] / NN - m * m
    scale = (1.0 / jnp.sqrt(v + 1e-5)) * g[...]
    yv = y[...]
    bv = b[...]
    for c in range(Cout):
      s = slice(c * 128, (c + 1) * 128)
      outs[c][...] = (yv[:, s] - m[:, s]) * scale[:, s] + bv[:, s]

  in_specs = [pl.BlockSpec((NB, Fout), lambda i: (i, 0)),
              pl.BlockSpec((1, Fout), lambda i: (0, 0)),
              pl.BlockSpec((1, Fout), lambda i: (0, 0)),
              pl.BlockSpec((1, Fout), lambda i: (0, 0)),
              pl.BlockSpec((1, Fout), lambda i: (0, 0))]
  out_specs = [pl.BlockSpec((NB, 128), lambda i: (i, 0)) for _ in range(Cout)]
  outs = [jax.ShapeDtypeStruct((NN, 128), F32) for _ in range(Cout)]
  return pl.pallas_call(body, grid=(n,), in_specs=in_specs,
                        out_specs=out_specs, out_shape=outs)


_norm_1 = _make_norm(512)
_norm_2 = _make_norm(1024)


def _final_body(y3, batchr, ssum, ssq, g3, b3, Wf1, bf1, Wf2, bf2, Wf3, bf3,
                out, Sacc, Cacc):
  i = pl.program_id(0)

  @pl.when(i == 0)
  def _():
    Sacc[...] = jnp.zeros((GG, 2048), F32)
    Cacc[...] = jnp.zeros((GG, 128), F32)

  bvec = batchr[0, 0, :]
  gid = lax.broadcasted_iota(jnp.int32, (GG, NB), 0)
  M = (bvec[None, :] == gid).astype(F32)
  Sacc[...] += _dotx(M, y3[...])
  Cacc[...] += jnp.sum(M, axis=1, keepdims=True)

  @pl.when(i == (NN // NB) - 1)
  def _():
    m = ssum[...] / NN
    v = ssq[...] / NN - m * m
    r = 1.0 / jnp.sqrt(v + 1e-5)
    cnt = Cacc[:, 0:1]
    ph = (Sacc[...] - cnt * m) * (r * g3[...]) + cnt * b3[...]
    pooled = ph / jnp.maximum(cnt, 1.0)
    t = jnp.maximum(_dot(pooled, Wf1[...]) + bf1[...], 0.0)
    t = jnp.maximum(_dot(t, Wf2[...]) + bf2[...], 0.0)
    out[...] = _dot(t, Wf3[...]) + bf3[...]


def _final(y3, batchr, s3, q3, g3r, b3r, Wf1, bf1r, Wf2, bf2r, Wf3, bf3r):
  n = NN // NB
  in_specs = [pl.BlockSpec((NB, 2048), lambda i: (i, 0)),
              pl.BlockSpec((1, 1, NB), lambda i: (i, 0, 0))] + [
      pl.BlockSpec(s, lambda i: (0, 0))
      for s in [(1, 2048), (1, 2048), (1, 2048), (1, 2048),
                (2048, 1024), (1, 1024), (1024, 512), (1, 512),
                (512, 86), (1, 86)]
  ]
  return pl.pallas_call(
      _final_body,
      grid=(n,),
      in_specs=in_specs,
      out_specs=pl.BlockSpec((GG, 86), lambda i: (0, 0)),
      out_shape=jax.ShapeDtypeStruct((GG, 86), F32),
      scratch_shapes=[pltpu.VMEM((GG, 2048), F32), pltpu.VMEM((GG, 128), F32)],
  )(y3, batchr, s3, q3, g3r, b3r, Wf1, bf1r, Wf2, bf2r, Wf3, bf3r)


def kernel(x, edge_index, edge_attr, batch, Wem1, bem1, Wem2, bem2, We1, be1,
           Wn1, nb1, g1, b1, We2, be2, Wn2, nb2, g2, b2, We3, be3, Wn3, nb3,
           g3, b3, Wf1, bf1, Wf2, bf2, Wf3, bf3):
  src = edge_index[0]
  dst = edge_index[1]
  r = lambda v: v.reshape(1, -1)

  ea_p = jnp.pad(edge_attr, ((0, 0), (0, 2)))
  Wem1p = jnp.pad(Wem1, ((0, 2), (0, 0)))
  We1p = jnp.pad(We1, ((0, 0), (0, 122)))
  be1p = jnp.pad(be1, (0, 122))
  x_p = jnp.pad(x, ((0, 0), (0, 122)))
  Wn1p = jnp.pad(Wn1, ((0, 122), (0, 0)))
  batchr = batch.reshape(NN // NB, 1, NB)

  e1, e2c, e3c = _edge(ea_p, Wem1p, r(bem1), Wem2, r(bem2), We1p, r(be1p),
                       We2, r(be2), We3, r(be3))

  p1 = _sc_stage_1(x_p, e1, src, dst)
  y1, s1, q1 = _node_1(x_p, p1[0], Wn1p, r(nb1))
  h1c = _norm_1(y1, s1, q1, r(g1), r(b1))

  p2 = _sc_stage_2(*h1c, *e2c, src, dst)
  y2, s2, q2 = _node_2(*h1c, *p2, Wn2, r(nb2))
  h2c = _norm_2(y2, s2, q2, r(g2), r(b2))

  p3 = _sc_stage_3(*h2c, *e3c, src, dst)
  y3, s3, q3 = _node_3(*h2c, *p3, Wn3, r(nb3))

  return _final(y3, batchr, s3, q3, r(g3), r(b3), Wf1, r(bf1), Wf2, r(bf2),
                Wf3, r(bf3))


# speedup vs baseline: 2.3850x; 1.0223x over previous
import functools

import jax
import jax.numpy as jnp
from jax import lax
from jax.experimental import pallas as pl
from jax.experimental.pallas import tpu as pltpu
from jax.experimental.pallas import tpu_sc as plsc

NN = 10000
EE = 160000
GG = 64
NB = 400
EB = 2000
SB = 64
NTILE = 32
STRIPE = 640
ZR = 80
F32 = jnp.float32


def _dot(a, b):
  return jnp.dot(a.astype(jnp.bfloat16), b.astype(jnp.bfloat16),
                 preferred_element_type=F32)


def _dotx(a, b):
  return jnp.dot(a, b, preferred_element_type=F32,
                 precision=lax.Precision.HIGHEST)


def _make_edge(Fout):
  C = Fout // 128

  def body(ea, Wm, bm, Wm2, bm2, Wl, bl, *eouts):
    a1 = jnp.maximum(_dot(ea[...], Wm[...]) + bm[...], 0.0)
    a = _dot(a1, Wm2[...]) + bm2[...]
    Wv = Wl[...]
    bv = bl[...]
    for c in range(C):
      s = slice(c * 128, (c + 1) * 128)
      eouts[c][...] = _dot(a, Wv[:, s]) + bv[:, s]

  n = EE // EB
  fullw = [
      pl.BlockSpec(s, lambda i: (0, 0))
      for s in [(8, 64), (1, 64), (64, 64), (1, 64), (64, Fout), (1, Fout)]
  ]
  outs = [jax.ShapeDtypeStruct((EE, 128), F32) for _ in range(C)]
  out_specs = [pl.BlockSpec((EB, 128), lambda i: (i, 0)) for _ in range(C)]
  return pl.pallas_call(
      body,
      grid=(n,),
      in_specs=[pl.BlockSpec((EB, 8), lambda i: (i, 0))] + fullw,
      out_specs=out_specs,
      out_shape=outs,
  )


_edge_1 = _make_edge(128)
_edge_2 = _make_edge(512)
_edge_3 = _make_edge(1024)


def _make_sc_stage(C, W):
  mesh = plsc.VectorSubcoreMesh(core_axis_name="c", subcore_axis_name="s")
  nblocks = EE // SB
  out_type = [jax.ShapeDtypeStruct((2, NN, W), F32) for _ in range(C)]
  bufset = [
      pltpu.VMEM((SB,), jnp.int32),
      pltpu.VMEM((SB,), jnp.int32),
      pltpu.VMEM((SB, W), F32),
      pltpu.VMEM((SB, W), F32),
      pltpu.SemaphoreType.DMA,
      pltpu.SemaphoreType.DMA,
  ]
  scratch = [
      pltpu.VMEM_SHARED((NN, W), F32),
  ] + bufset + bufset

  @functools.partial(pl.kernel, out_type=out_type, mesh=mesh,
                     scratch_types=scratch)
  def k(*refs):
    h_refs = refs[:C]
    e_refs = refs[C:2 * C]
    src_hbm = refs[2 * C]
    dst_hbm = refs[2 * C + 1]
    out_refs = refs[2 * C + 2:3 * C + 2]
    slab = refs[3 * C + 2]
    bufA = refs[3 * C + 3:3 * C + 9]
    bufB = refs[3 * C + 9:3 * C + 15]
    core = lax.axis_index("c")
    sub = lax.axis_index("s")
    wid = core * 16 + sub

    zvec = jnp.zeros((16,), F32)

    nb_t = (nblocks - wid + NTILE - 1) // NTILE
    blk_off = lambda j: (wid + j * NTILE) * SB

    def start_idx(S, off):
      pltpu.make_async_copy(src_hbm.at[pl.ds(off, SB)], S[0], S[4]).start()
      pltpu.make_async_copy(dst_hbm.at[pl.ds(off, SB)], S[1], S[4]).start()

    def wait_idx(S):
      pltpu.make_async_copy(src_hbm.at[pl.ds(0, SB)], S[0], S[4]).wait()
      pltpu.make_async_copy(dst_hbm.at[pl.ds(0, SB)], S[1], S[4]).wait()

    def start_ge(S, c, off):
      pltpu.make_async_copy(h_refs[c].at[S[0]], S[2], S[5]).start()
      pltpu.make_async_copy(e_refs[c].at[pl.ds(off, SB)], S[3], S[5]).start()

    def wait_ge(S, c):
      pltpu.make_async_copy(h_refs[c].at[S[0]], S[2], S[5]).wait()
      pltpu.make_async_copy(e_refs[c].at[pl.ds(0, SB)], S[3], S[5]).wait()

    base = sub * STRIPE
    nzero = jnp.where(sub == 15, 6, 10)

    for c in range(C):
      def zr(rr, carry):
        for kk in range(W // 16):
          bufA[2][rr, pl.ds(kk * 16, 16)] = zvec
        return carry

      lax.fori_loop(0, SB, zr, 0)

      def zb(kk, carry):
        pltpu.make_async_copy(bufA[2], slab.at[pl.ds(base + kk * SB, SB)],
                              bufA[4]).start()
        return carry

      lax.fori_loop(0, nzero, zb, 0)

      @pl.when(sub == 15)
      def _():
        pltpu.make_async_copy(bufA[2].at[pl.ds(0, 16)],
                              slab.at[pl.ds(base + 384, 16)], bufA[4]).start()

      def zw(kk, carry):
        pltpu.make_async_copy(bufA[2], slab.at[pl.ds(base, SB)],
                              bufA[4]).wait()
        return carry

      lax.fori_loop(0, nzero, zw, 0)

      @pl.when(sub == 15)
      def _():
        pltpu.make_async_copy(bufA[2].at[pl.ds(0, 16)],
                              slab.at[pl.ds(base, 16)], bufA[4]).wait()

      plsc.subcore_barrier()

      start_idx(bufA, blk_off(0))
      wait_idx(bufA)
      start_ge(bufA, c, blk_off(0))

      @pl.when(1 < nb_t)
      def _():
        start_idx(bufB, blk_off(1))

      def step(j, X, Y):
        wait_ge(X, c)

        @pl.when(j + 1 < nb_t)
        def _():
          wait_idx(Y)
          start_ge(Y, c, blk_off(j + 1))

        def rb(rr, rc):
          for dd in range(2):
            for kk in range(W // 16):
              s = pl.ds(kk * 16, 16)
              X[2][rr * 2 + dd, s] = jnp.maximum(
                  X[2][rr * 2 + dd, s] + X[3][rr * 2 + dd, s], 0.0)
          return rc

        lax.fori_loop(0, SB // 2, rb, 0)
        pltpu.sync_copy(X[2], slab.at[X[1]], add=True)

        @pl.when(j + 2 < nb_t)
        def _():
          start_idx(X, blk_off(j + 2))

      def eb(j, carry):

        @pl.when(j % 2 == 0)
        def _():
          step(j, bufA, bufB)

        @pl.when(j % 2 == 1)
        def _():
          step(j, bufB, bufA)

        return carry

      lax.fori_loop(0, nb_t, eb, 0)
      plsc.subcore_barrier()

      @pl.when(sub < 15)
      def _():
        pltpu.sync_copy(slab.at[pl.ds(base, STRIPE)],
                        out_refs[c].at[core, pl.ds(base, STRIPE)])

      @pl.when(sub == 15)
      def _():
        pltpu.sync_copy(slab.at[pl.ds(base, NN - 15 * STRIPE)],
                        out_refs[c].at[core, pl.ds(base, NN - 15 * STRIPE)])

  return k


_sc_stage_1 = _make_sc_stage(1, 128)
_sc_stage_2 = _make_sc_stage(4, 128)
_sc_stage_3 = _make_sc_stage(8, 128)


def _make_node(Cin, Win, Fout):
  n = NN // NB

  def body(*refs):
    h = refs[:Cin]
    p = refs[Cin:2 * Cin]
    Wn = refs[2 * Cin]
    nb_ = refs[2 * Cin + 1]
    y, ssum, ssq = refs[2 * Cin + 2:]
    i = pl.program_id(0)
    Wv = Wn[...]
    z = jnp.broadcast_to(nb_[...], (NB, Fout))
    for c in range(Cin):
      pv = p[c][...]
      hc = h[c][...] + pv[0] + pv[1]
      z = z + _dot(hc, Wv[c * Win:(c + 1) * Win, :])
    yv = jnp.maximum(z, 0.0)
    y[...] = yv

    @pl.when(i == 0)
    def _():
      ssum[...] = jnp.zeros((1, Fout), F32)
      ssq[...] = jnp.zeros((1, Fout), F32)

    ssum[...] += jnp.sum(yv, axis=0, keepdims=True)
    ssq[...] += jnp.sum(yv * yv, axis=0, keepdims=True)

  in_specs = ([pl.BlockSpec((NB, Win), lambda i: (i, 0)) for _ in range(Cin)] +
              [pl.BlockSpec((2, NB, Win), lambda i: (0, i, 0)) for _ in range(Cin)] +
              [pl.BlockSpec((Cin * Win, Fout), lambda i: (0, 0)),
               pl.BlockSpec((1, Fout), lambda i: (0, 0))])
  out_specs = [pl.BlockSpec((NB, Fout), lambda i: (i, 0)),
               pl.BlockSpec((1, Fout), lambda i: (0, 0)),
               pl.BlockSpec((1, Fout), lambda i: (0, 0))]
  outs = [jax.ShapeDtypeStruct((NN, Fout), F32),
          jax.ShapeDtypeStruct((1, Fout), F32),
          jax.ShapeDtypeStruct((1, Fout), F32)]
  return pl.pallas_call(body, grid=(n,), in_specs=in_specs,
                        out_specs=out_specs, out_shape=outs)


_node_1 = _make_node(1, 128, 512)
_node_2 = _make_node(4, 128, 1024)
_node_3 = _make_node(8, 128, 2048)


def _make_norm(Fout):
  Cout = Fout // 128
  n = NN // NB

  def body(y, ssum, ssq, g, b, *outs):
    m = ssum[...] / NN
    v = ssq[...] / NN - m * m
    scale = (1.0 / jnp.sqrt(v + 1e-5)) * g[...]
    yv = y[...]
    bv = b[...]
    for c in range(Cout):
      s = slice(c * 128, (c + 1) * 128)
      outs[c][...] = (yv[:, s] - m[:, s]) * scale[:, s] + bv[:, s]

  in_specs = [pl.BlockSpec((NB, Fout), lambda i: (i, 0)),
              pl.BlockSpec((1, Fout), lambda i: (0, 0)),
              pl.BlockSpec((1, Fout), lambda i: (0, 0)),
              pl.BlockSpec((1, Fout), lambda i: (0, 0)),
              pl.BlockSpec((1, Fout), lambda i: (0, 0))]
  out_specs = [pl.BlockSpec((NB, 128), lambda i: (i, 0)) for _ in range(Cout)]
  outs = [jax.ShapeDtypeStruct((NN, 128), F32) for _ in range(Cout)]
  return pl.pallas_call(body, grid=(n,), in_specs=in_specs,
                        out_specs=out_specs, out_shape=outs)


_norm_1 = _make_norm(512)
_norm_2 = _make_norm(1024)


def _final_body(y3, batchr, ssum, ssq, g3, b3, Wf1, bf1, Wf2, bf2, Wf3, bf3,
                out, Sacc, Cacc):
  i = pl.program_id(0)

  @pl.when(i == 0)
  def _():
    Sacc[...] = jnp.zeros((GG, 2048), F32)
    Cacc[...] = jnp.zeros((GG, 128), F32)

  bvec = batchr[0, 0, :]
  gid = lax.broadcasted_iota(jnp.int32, (GG, NB), 0)
  M = (bvec[None, :] == gid).astype(F32)
  Sacc[...] += _dotx(M, y3[...])
  Cacc[...] += jnp.sum(M, axis=1, keepdims=True)

  @pl.when(i == (NN // NB) - 1)
  def _():
    m = ssum[...] / NN
    v = ssq[...] / NN - m * m
    r = 1.0 / jnp.sqrt(v + 1e-5)
    cnt = Cacc[:, 0:1]
    ph = (Sacc[...] - cnt * m) * (r * g3[...]) + cnt * b3[...]
    pooled = ph / jnp.maximum(cnt, 1.0)
    t = jnp.maximum(_dot(pooled, Wf1[...]) + bf1[...], 0.0)
    t = jnp.maximum(_dot(t, Wf2[...]) + bf2[...], 0.0)
    out[...] = _dot(t, Wf3[...]) + bf3[...]


def _final(y3, batchr, s3, q3, g3r, b3r, Wf1, bf1r, Wf2, bf2r, Wf3, bf3r):
  n = NN // NB
  in_specs = [pl.BlockSpec((NB, 2048), lambda i: (i, 0)),
              pl.BlockSpec((1, 1, NB), lambda i: (i, 0, 0))] + [
      pl.BlockSpec(s, lambda i: (0, 0))
      for s in [(1, 2048), (1, 2048), (1, 2048), (1, 2048),
                (2048, 1024), (1, 1024), (1024, 512), (1, 512),
                (512, 86), (1, 86)]
  ]
  return pl.pallas_call(
      _final_body,
      grid=(n,),
      in_specs=in_specs,
      out_specs=pl.BlockSpec((GG, 86), lambda i: (0, 0)),
      out_shape=jax.ShapeDtypeStruct((GG, 86), F32),
      scratch_shapes=[pltpu.VMEM((GG, 2048), F32), pltpu.VMEM((GG, 128), F32)],
  )(y3, batchr, s3, q3, g3r, b3r, Wf1, bf1r, Wf2, bf2r, Wf3, bf3r)


def kernel(x, edge_index, edge_attr, batch, Wem1, bem1, Wem2, bem2, We1, be1,
           Wn1, nb1, g1, b1, We2, be2, Wn2, nb2, g2, b2, We3, be3, Wn3, nb3,
           g3, b3, Wf1, bf1, Wf2, bf2, Wf3, bf3):
  src = edge_index[0]
  dst = edge_index[1]
  r = lambda v: v.reshape(1, -1)

  ea_p = jnp.pad(edge_attr, ((0, 0), (0, 2)))
  Wem1p = jnp.pad(Wem1, ((0, 2), (0, 0)))
  We1p = jnp.pad(We1, ((0, 0), (0, 122)))
  be1p = jnp.pad(be1, (0, 122))
  x_p = jnp.pad(x, ((0, 0), (0, 122)))
  Wn1p = jnp.pad(Wn1, ((0, 122), (0, 0)))
  batchr = batch.reshape(NN // NB, 1, NB)

  e1 = _edge_1(ea_p, Wem1p, r(bem1), Wem2, r(bem2), We1p, r(be1p))[0]
  p1 = _sc_stage_1(x_p, e1, src, dst)
  e2c = _edge_2(ea_p, Wem1p, r(bem1), Wem2, r(bem2), We2, r(be2))
  e3c = _edge_3(ea_p, Wem1p, r(bem1), Wem2, r(bem2), We3, r(be3))
  y1, s1, q1 = _node_1(x_p, p1[0], Wn1p, r(nb1))
  h1c = _norm_1(y1, s1, q1, r(g1), r(b1))

  p2 = _sc_stage_2(*h1c, *e2c, src, dst)
  y2, s2, q2 = _node_2(*h1c, *p2, Wn2, r(nb2))
  h2c = _norm_2(y2, s2, q2, r(g2), r(b2))

  p3 = _sc_stage_3(*h2c, *e3c, src, dst)
  y3, s3, q3 = _node_3(*h2c, *p3, Wn3, r(nb3))

  return _final(y3, batchr, s3, q3, r(g3), r(b3), Wf1, r(bf1), Wf2, r(bf2),
                Wf3, r(bf3))


# speedup vs baseline: 2.4461x; 1.0256x over previous
import functools

import jax
import jax.numpy as jnp
from jax import lax
from jax.experimental import pallas as pl
from jax.experimental.pallas import tpu as pltpu
from jax.experimental.pallas import tpu_sc as plsc

NN = 10000
EE = 160000
GG = 64
NB = 400
EB = 2000
SB = 64
NTILE = 32
STRIPE = 640
ZR = 80
F32 = jnp.float32


def _dot(a, b):
  return jnp.dot(a.astype(jnp.bfloat16), b.astype(jnp.bfloat16),
                 preferred_element_type=F32)


def _dotx(a, b):
  return jnp.dot(a, b, preferred_element_type=F32,
                 precision=lax.Precision.HIGHEST)


def _make_edge(Fout):
  C = Fout // 128

  def body(ea, Wm, bm, Wm2, bm2, Wl, bl, *eouts):
    a1 = jnp.maximum(_dot(ea[...], Wm[...]) + bm[...], 0.0)
    a = _dot(a1, Wm2[...]) + bm2[...]
    Wv = Wl[...]
    bv = bl[...]
    for c in range(C):
      s = slice(c * 128, (c + 1) * 128)
      eouts[c][...] = _dot(a, Wv[:, s]) + bv[:, s]

  n = EE // EB
  fullw = [
      pl.BlockSpec(s, lambda i: (0, 0))
      for s in [(8, 64), (1, 64), (64, 64), (1, 64), (64, Fout), (1, Fout)]
  ]
  outs = [jax.ShapeDtypeStruct((EE, 128), F32) for _ in range(C)]
  out_specs = [pl.BlockSpec((EB, 128), lambda i: (i, 0)) for _ in range(C)]
  return pl.pallas_call(
      body,
      grid=(n,),
      in_specs=[pl.BlockSpec((EB, 8), lambda i: (i, 0))] + fullw,
      out_specs=out_specs,
      out_shape=outs,
  )


_edge_1 = _make_edge(128)
_edge_2 = _make_edge(512)
_edge_3 = _make_edge(1024)


def _make_sc_stage(C, W):
  mesh = plsc.VectorSubcoreMesh(core_axis_name="c", subcore_axis_name="s")
  nblocks = EE // SB
  out_type = [jax.ShapeDtypeStruct((2, NN, W), F32) for _ in range(C)]
  bufset = [
      pltpu.VMEM((SB,), jnp.int32),
      pltpu.VMEM((SB,), jnp.int32),
      pltpu.VMEM((SB, W), F32),
      pltpu.VMEM((SB, W), F32),
      pltpu.SemaphoreType.DMA,
      pltpu.SemaphoreType.DMA,
      pltpu.VMEM((SB,), jnp.int32),
      pltpu.SemaphoreType.DMA,
  ]
  scratch = [
      pltpu.VMEM_SHARED((NN, W), F32),
  ] + bufset + bufset

  @functools.partial(pl.kernel, out_type=out_type, mesh=mesh,
                     scratch_types=scratch)
  def k(*refs):
    h_refs = refs[:C]
    e_refs = refs[C:2 * C]
    src_hbm = refs[2 * C]
    dst_hbm = refs[2 * C + 1]
    out_refs = refs[2 * C + 2:3 * C + 2]
    slab = refs[3 * C + 2]
    bufA = refs[3 * C + 3:3 * C + 11]
    bufB = refs[3 * C + 11:3 * C + 19]
    core = lax.axis_index("c")
    sub = lax.axis_index("s")
    wid = core * 16 + sub

    zvec = jnp.zeros((16,), F32)

    nb_t = (nblocks - wid + NTILE - 1) // NTILE
    blk_off = lambda j: (wid + j * NTILE) * SB

    def start_idx(S, off):
      pltpu.make_async_copy(src_hbm.at[pl.ds(off, SB)], S[0], S[4]).start()
      pltpu.make_async_copy(dst_hbm.at[pl.ds(off, SB)], S[1], S[4]).start()

    def wait_idx(S):
      pltpu.make_async_copy(src_hbm.at[pl.ds(0, SB)], S[0], S[4]).wait()
      pltpu.make_async_copy(dst_hbm.at[pl.ds(0, SB)], S[1], S[4]).wait()

    def start_ge(S, c, off):
      pltpu.make_async_copy(h_refs[c].at[S[0]], S[2], S[5]).start()
      pltpu.make_async_copy(e_refs[c].at[pl.ds(off, SB)], S[3], S[5]).start()

    def wait_ge(S, c):
      pltpu.make_async_copy(h_refs[c].at[S[0]], S[2], S[5]).wait()
      pltpu.make_async_copy(e_refs[c].at[pl.ds(0, SB)], S[3], S[5]).wait()

    def wait_scatter(S):
      pltpu.make_async_copy(S[2], slab.at[S[6]], S[7]).wait()

    base = sub * STRIPE
    nzero = jnp.where(sub == 15, 6, 10)

    for c in range(C):
      def zr(rr, carry):
        for kk in range(W // 16):
          bufA[2][rr, pl.ds(kk * 16, 16)] = zvec
        return carry

      lax.fori_loop(0, SB, zr, 0)

      def zb(kk, carry):
        pltpu.make_async_copy(bufA[2], slab.at[pl.ds(base + kk * SB, SB)],
                              bufA[4]).start()
        return carry

      lax.fori_loop(0, nzero, zb, 0)

      @pl.when(sub == 15)
      def _():
        pltpu.make_async_copy(bufA[2].at[pl.ds(0, 16)],
                              slab.at[pl.ds(base + 384, 16)], bufA[4]).start()

      def zw(kk, carry):
        pltpu.make_async_copy(bufA[2], slab.at[pl.ds(base, SB)],
                              bufA[4]).wait()
        return carry

      lax.fori_loop(0, nzero, zw, 0)

      @pl.when(sub == 15)
      def _():
        pltpu.make_async_copy(bufA[2].at[pl.ds(0, 16)],
                              slab.at[pl.ds(base, 16)], bufA[4]).wait()

      plsc.subcore_barrier()

      start_idx(bufA, blk_off(0))
      wait_idx(bufA)
      start_ge(bufA, c, blk_off(0))

      @pl.when(1 < nb_t)
      def _():
        start_idx(bufB, blk_off(1))

      def step(j, X, Y):
        wait_ge(X, c)

        @pl.when(j + 1 < nb_t)
        def _():
          wait_idx(Y)

          @pl.when(j >= 1)
          def _():
            wait_scatter(Y)

          start_ge(Y, c, blk_off(j + 1))

        def rb(rr, rc):
          for dd in range(2):
            for kk in range(W // 16):
              s = pl.ds(kk * 16, 16)
              X[2][rr * 2 + dd, s] = jnp.maximum(
                  X[2][rr * 2 + dd, s] + X[3][rr * 2 + dd, s], 0.0)
          return rc

        lax.fori_loop(0, SB // 2, rb, 0)
        for kk in range(SB // 16):
          s = pl.ds(kk * 16, 16)
          X[6][s] = X[1][s]
        pltpu.async_copy(X[2], slab.at[X[6]], X[7], add=True)

        @pl.when(j + 2 < nb_t)
        def _():
          start_idx(X, blk_off(j + 2))

      def eb(j, carry):

        @pl.when(j % 2 == 0)
        def _():
          step(j, bufA, bufB)

        @pl.when(j % 2 == 1)
        def _():
          step(j, bufB, bufA)

        return carry

      lax.fori_loop(0, nb_t, eb, 0)
      wait_scatter(bufA)
      wait_scatter(bufB)
      plsc.subcore_barrier()

      @pl.when(sub < 15)
      def _():
        pltpu.sync_copy(slab.at[pl.ds(base, STRIPE)],
                        out_refs[c].at[core, pl.ds(base, STRIPE)])

      @pl.when(sub == 15)
      def _():
        pltpu.sync_copy(slab.at[pl.ds(base, NN - 15 * STRIPE)],
                        out_refs[c].at[core, pl.ds(base, NN - 15 * STRIPE)])

  return k


_sc_stage_1 = _make_sc_stage(1, 128)
_sc_stage_2 = _make_sc_stage(4, 128)
_sc_stage_3 = _make_sc_stage(8, 128)


def _make_node(Cin, Win, Fout):
  n = NN // NB

  def body(*refs):
    h = refs[:Cin]
    p = refs[Cin:2 * Cin]
    Wn = refs[2 * Cin]
    nb_ = refs[2 * Cin + 1]
    y, ssum, ssq = refs[2 * Cin + 2:]
    i = pl.program_id(0)
    Wv = Wn[...]
    z = jnp.broadcast_to(nb_[...], (NB, Fout))
    for c in range(Cin):
      pv = p[c][...]
      hc = h[c][...] + pv[0] + pv[1]
      z = z + _dot(hc, Wv[c * Win:(c + 1) * Win, :])
    yv = jnp.maximum(z, 0.0)
    y[...] = yv

    @pl.when(i == 0)
    def _():
      ssum[...] = jnp.zeros((1, Fout), F32)
      ssq[...] = jnp.zeros((1, Fout), F32)

    ssum[...] += jnp.sum(yv, axis=0, keepdims=True)
    ssq[...] += jnp.sum(yv * yv, axis=0, keepdims=True)

  in_specs = ([pl.BlockSpec((NB, Win), lambda i: (i, 0)) for _ in range(Cin)] +
              [pl.BlockSpec((2, NB, Win), lambda i: (0, i, 0)) for _ in range(Cin)] +
              [pl.BlockSpec((Cin * Win, Fout), lambda i: (0, 0)),
               pl.BlockSpec((1, Fout), lambda i: (0, 0))])
  out_specs = [pl.BlockSpec((NB, Fout), lambda i: (i, 0)),
               pl.BlockSpec((1, Fout), lambda i: (0, 0)),
               pl.BlockSpec((1, Fout), lambda i: (0, 0))]
  outs = [jax.ShapeDtypeStruct((NN, Fout), F32),
          jax.ShapeDtypeStruct((1, Fout), F32),
          jax.ShapeDtypeStruct((1, Fout), F32)]
  return pl.pallas_call(body, grid=(n,), in_specs=in_specs,
                        out_specs=out_specs, out_shape=outs)


_node_1 = _make_node(1, 128, 512)
_node_2 = _make_node(4, 128, 1024)
_node_3 = _make_node(8, 128, 2048)


def _make_norm(Fout):
  Cout = Fout // 128
  n = NN // NB

  def body(y, ssum, ssq, g, b, *outs):
    m = ssum[...] / NN
    v = ssq[...] / NN - m * m
    scale = (1.0 / jnp.sqrt(v + 1e-5)) * g[...]
    yv = y[...]
    bv = b[...]
    for c in range(Cout):
      s = slice(c * 128, (c + 1) * 128)
      outs[c][...] = (yv[:, s] - m[:, s]) * scale[:, s] + bv[:, s]

  in_specs = [pl.BlockSpec((NB, Fout), lambda i: (i, 0)),
              pl.BlockSpec((1, Fout), lambda i: (0, 0)),
              pl.BlockSpec((1, Fout), lambda i: (0, 0)),
              pl.BlockSpec((1, Fout), lambda i: (0, 0)),
              pl.BlockSpec((1, Fout), lambda i: (0, 0))]
  out_specs = [pl.BlockSpec((NB, 128), lambda i: (i, 0)) for _ in range(Cout)]
  outs = [jax.ShapeDtypeStruct((NN, 128), F32) for _ in range(Cout)]
  return pl.pallas_call(body, grid=(n,), in_specs=in_specs,
                        out_specs=out_specs, out_shape=outs)


_norm_1 = _make_norm(512)
_norm_2 = _make_norm(1024)


def _final_body(y3, batchr, ssum, ssq, g3, b3, Wf1, bf1, Wf2, bf2, Wf3, bf3,
                out, Sacc, Cacc):
  i = pl.program_id(0)

  @pl.when(i == 0)
  def _():
    Sacc[...] = jnp.zeros((GG, 2048), F32)
    Cacc[...] = jnp.zeros((GG, 128), F32)

  bvec = batchr[0, 0, :]
  gid = lax.broadcasted_iota(jnp.int32, (GG, NB), 0)
  M = (bvec[None, :] == gid).astype(F32)
  Sacc[...] += _dotx(M, y3[...])
  Cacc[...] += jnp.sum(M, axis=1, keepdims=True)

  @pl.when(i == (NN // NB) - 1)
  def _():
    m = ssum[...] / NN
    v = ssq[...] / NN - m * m
    r = 1.0 / jnp.sqrt(v + 1e-5)
    cnt = Cacc[:, 0:1]
    ph = (Sacc[...] - cnt * m) * (r * g3[...]) + cnt * b3[...]
    pooled = ph / jnp.maximum(cnt, 1.0)
    t = jnp.maximum(_dot(pooled, Wf1[...]) + bf1[...], 0.0)
    t = jnp.maximum(_dot(t, Wf2[...]) + bf2[...], 0.0)
    out[...] = _dot(t, Wf3[...]) + bf3[...]


def _final(y3, batchr, s3, q3, g3r, b3r, Wf1, bf1r, Wf2, bf2r, Wf3, bf3r):
  n = NN // NB
  in_specs = [pl.BlockSpec((NB, 2048), lambda i: (i, 0)),
              pl.BlockSpec((1, 1, NB), lambda i: (i, 0, 0))] + [
      pl.BlockSpec(s, lambda i: (0, 0))
      for s in [(1, 2048), (1, 2048), (1, 2048), (1, 2048),
                (2048, 1024), (1, 1024), (1024, 512), (1, 512),
                (512, 86), (1, 86)]
  ]
  return pl.pallas_call(
      _final_body,
      grid=(n,),
      in_specs=in_specs,
      out_specs=pl.BlockSpec((GG, 86), lambda i: (0, 0)),
      out_shape=jax.ShapeDtypeStruct((GG, 86), F32),
      scratch_shapes=[pltpu.VMEM((GG, 2048), F32), pltpu.VMEM((GG, 128), F32)],
  )(y3, batchr, s3, q3, g3r, b3r, Wf1, bf1r, Wf2, bf2r, Wf3, bf3r)


def kernel(x, edge_index, edge_attr, batch, Wem1, bem1, Wem2, bem2, We1, be1,
           Wn1, nb1, g1, b1, We2, be2, Wn2, nb2, g2, b2, We3, be3, Wn3, nb3,
           g3, b3, Wf1, bf1, Wf2, bf2, Wf3, bf3):
  src = edge_index[0]
  dst = edge_index[1]
  r = lambda v: v.reshape(1, -1)

  ea_p = jnp.pad(edge_attr, ((0, 0), (0, 2)))
  Wem1p = jnp.pad(Wem1, ((0, 2), (0, 0)))
  We1p = jnp.pad(We1, ((0, 0), (0, 122)))
  be1p = jnp.pad(be1, (0, 122))
  x_p = jnp.pad(x, ((0, 0), (0, 122)))
  Wn1p = jnp.pad(Wn1, ((0, 122), (0, 0)))
  batchr = batch.reshape(NN // NB, 1, NB)

  e1 = _edge_1(ea_p, Wem1p, r(bem1), Wem2, r(bem2), We1p, r(be1p))[0]
  p1 = _sc_stage_1(x_p, e1, src, dst)
  e2c = _edge_2(ea_p, Wem1p, r(bem1), Wem2, r(bem2), We2, r(be2))
  e3c = _edge_3(ea_p, Wem1p, r(bem1), Wem2, r(bem2), We3, r(be3))
  y1, s1, q1 = _node_1(x_p, p1[0], Wn1p, r(nb1))
  h1c = _norm_1(y1, s1, q1, r(g1), r(b1))

  p2 = _sc_stage_2(*h1c, *e2c, src, dst)
  y2, s2, q2 = _node_2(*h1c, *p2, Wn2, r(nb2))
  h2c = _norm_2(y2, s2, q2, r(g2), r(b2))

  p3 = _sc_stage_3(*h2c, *e3c, src, dst)
  y3, s3, q3 = _node_3(*h2c, *p3, Wn3, r(nb3))

  return _final(y3, batchr, s3, q3, r(g3), r(b3), Wf1, r(bf1), Wf2, r(bf2),
                Wf3, r(bf3))
